# baseline (speedup 1.0000x reference)
import functools

import jax
import jax.numpy as jnp
from jax import lax
from jax.experimental import pallas as pl
from jax.experimental.pallas import tpu as pltpu
from jax.experimental.pallas import tpu_sc as plsc

N = 10000
NP = 10240
E = 320000
D_IN = 128
HID = 256
HALF = 128

EC = 128
NSUB = 16
RPS = NP // NSUB
BLK = 16
NBLK = 10
CPS = BLK * NBLK
EP = CPS * NSUB * EC
CHT = EP // EC

_MESH = plsc.VectorSubcoreMesh(core_axis_name="c", subcore_axis_name="s")


@functools.partial(
    pl.kernel,
    out_type=(
        jax.ShapeDtypeStruct((NP, HALF), jnp.float32),
        jax.ShapeDtypeStruct((NP, HALF), jnp.float32),
    ),
    mesh=_MESH,
    scratch_types=(
        pltpu.VMEM((2, EC), jnp.int32),
        pltpu.VMEM((2, EC), jnp.int32),
        pltpu.VMEM((EC, HALF), jnp.float32),
        pltpu.VMEM((EC, HALF), jnp.float32),
        pltpu.VMEM_SHARED((NP, HALF), jnp.float32),
        pltpu.SemaphoreType.DMA,
    ),
)
def _sc_edge_sum(tab_l, tab_r, src2d, dst2d, zrows,
                 out_l, out_r, si2, di2, rows0, rows1, acc, gsem):
    c = lax.axis_index("c")
    s = lax.axis_index("s")
    slab = pl.ds(s * RPS, RPS)
    pltpu.sync_copy(zrows, acc.at[slab])
    plsc.subcore_barrier()

    def accumulate(tab):
        def step(i, carry):
            base = s * CPS + i * 2
            pltpu.sync_copy(src2d.at[pl.ds(base, 2)], si2)
            pltpu.sync_copy(dst2d.at[pl.ds(base, 2)], di2)
            cg0 = pltpu.async_copy(tab.at[si2.at[0]], rows0, gsem)
            cg1 = pltpu.async_copy(tab.at[si2.at[1]], rows1, gsem)
            cg0.wait()
            cg1.wait()
            pltpu.sync_copy(rows0, acc.at[di2.at[0]], add=True)
            pltpu.sync_copy(rows1, acc.at[di2.at[1]], add=True)
            return carry

        lax.fori_loop(0, CPS // 2, step, 0)

    @pl.when(c == 0)
    def _():
        accumulate(tab_l)

    @pl.when(c == 1)
    def _():
        accumulate(tab_r)

    plsc.subcore_barrier()

    @pl.when(c == 0)
    def _():
        pltpu.sync_copy(acc.at[slab], out_l.at[slab])

    @pl.when(c == 1)
    def _():
        pltpu.sync_copy(acc.at[slab], out_r.at[slab])


@functools.partial(
    pl.kernel,
    out_type=(
        jax.ShapeDtypeStruct((NP, HALF), jnp.float32),
        jax.ShapeDtypeStruct((NP, HALF), jnp.float32),
    ),
    mesh=_MESH,
    scratch_types=(
        pltpu.VMEM((BLK, EC), jnp.int32),
        pltpu.VMEM((EC, HALF), jnp.float32),
        pltpu.VMEM_SHARED((NP, HALF), jnp.float32),
        pltpu.SemaphoreType.DMA,
    ),
)
def _sc_edge_counts(dst_sa, dst_as, zcnt, ones_hbm,
                    cnt_a, cnt_s, di_blk, ones_v, csh, sem):
    c = lax.axis_index("c")
    s = lax.axis_index("s")
    slab = pl.ds(s * RPS, RPS)
    pltpu.sync_copy(ones_hbm, ones_v)
    pltpu.sync_copy(zcnt, csh.at[slab])
    plsc.subcore_barrier()

    def accumulate(dst2d):
        def block(bi, carry):
            base = (s * NBLK + bi) * BLK
            pltpu.sync_copy(dst2d.at[pl.ds(base, BLK)], di_blk)
            cs = [pltpu.async_copy(ones_v, csh.at[di_blk.at[k]], sem,
                                   add=True)
                  for k in range(BLK)]
            for cp in cs:
                cp.wait()
            return carry

        lax.fori_loop(0, NBLK, block, 0)

    @pl.when(c == 0)
    def _():
        accumulate(dst_sa)

    @pl.when(c == 1)
    def _():
        accumulate(dst_as)

    plsc.subcore_barrier()

    @pl.when(c == 0)
    def _():
        pltpu.sync_copy(csh.at[slab], cnt_a.at[slab])

    @pl.when(c == 1)
    def _():
        pltpu.sync_copy(csh.at[slab], cnt_s.at[slab])


_B = 2048


def _dot(a, b):
    return jnp.dot(a.astype(jnp.bfloat16), b.astype(jnp.bfloat16),
                   preferred_element_type=jnp.float32)


def _inproj_body(x_ref, w_ref, b_ref, ol_ref, or_ref):
    h = jnp.maximum(_dot(x_ref[...], w_ref[...]) + b_ref[...], 0.0)
    ol_ref[...] = h[:, :HALF]
    or_ref[...] = h[:, HALF:]


_inproj = pl.pallas_call(
    _inproj_body,
    grid=(NP // _B,),
    in_specs=[
        pl.BlockSpec((_B, D_IN), lambda i: (i, 0)),
        pl.BlockSpec((D_IN, HID), lambda i: (0, 0)),
        pl.BlockSpec((1, HID), lambda i: (0, 0)),
    ],
    out_specs=[
        pl.BlockSpec((_B, HALF), lambda i: (i, 0)),
        pl.BlockSpec((_B, HALF), lambda i: (i, 0)),
    ],
    out_shape=[
        jax.ShapeDtypeStruct((NP, HALF), jnp.float32),
        jax.ShapeDtypeStruct((NP, HALF), jnp.float32),
    ],
)


def _combine_body(al_ref, ar_ref, cnt_ref, hl_ref, hr_ref,
                  wl_ref, bl_ref, wr_ref, g_ref, b_ref, ol_ref, or_ref):
    agg = jnp.concatenate([al_ref[...], ar_ref[...]], axis=1)
    h = jnp.concatenate([hl_ref[...], hr_ref[...]], axis=1)
    mean = agg / jnp.maximum(cnt_ref[...][:, :1], 1.0)
    new = _dot(mean, wl_ref[...]) + _dot(h, wr_ref[...]) + bl_ref[...]
    m = jnp.mean(new, axis=1, keepdims=True)
    v = jnp.mean((new - m) * (new - m), axis=1, keepdims=True)
    y = (new - m) / jnp.sqrt(v + 1e-5) * g_ref[...] + b_ref[...]
    y = jnp.maximum(y, 0.0)
    ol_ref[...] = y[:, :HALF]
    or_ref[...] = y[:, HALF:]


_combine = pl.pallas_call(
    _combine_body,
    grid=(NP // _B,),
    in_specs=[
        pl.BlockSpec((_B, HALF), lambda i: (i, 0)),
        pl.BlockSpec((_B, HALF), lambda i: (i, 0)),
        pl.BlockSpec((_B, HALF), lambda i: (i, 0)),
        pl.BlockSpec((_B, HALF), lambda i: (i, 0)),
        pl.BlockSpec((_B, HALF), lambda i: (i, 0)),
        pl.BlockSpec((HID, HID), lambda i: (0, 0)),
        pl.BlockSpec((1, HID), lambda i: (0, 0)),
        pl.BlockSpec((HID, HID), lambda i: (0, 0)),
        pl.BlockSpec((1, HID), lambda i: (0, 0)),
        pl.BlockSpec((1, HID), lambda i: (0, 0)),
    ],
    out_specs=[
        pl.BlockSpec((_B, HALF), lambda i: (i, 0)),
        pl.BlockSpec((_B, HALF), lambda i: (i, 0)),
    ],
    out_shape=[
        jax.ShapeDtypeStruct((NP, HALF), jnp.float32),
        jax.ShapeDtypeStruct((NP, HALF), jnp.float32),
    ],
)


def _head_body(hl_ref, hr_ref, w_ref, bias_ref, hid_ref, pred_ref):
    h = jnp.concatenate([hl_ref[...], hr_ref[...]], axis=1)
    hid_ref[...] = h
    wrow = jnp.reshape(w_ref[...], (1, HID))
    h16 = h.astype(jnp.bfloat16).astype(jnp.float32)
    w16 = wrow.astype(jnp.bfloat16).astype(jnp.float32)
    pred_ref[...] = (jnp.sum(h16 * w16, axis=1, keepdims=True)
                     + bias_ref[...])


_head = pl.pallas_call(
    _head_body,
    grid=(NP // _B,),
    in_specs=[
        pl.BlockSpec((_B, HALF), lambda i: (i, 0)),
        pl.BlockSpec((_B, HALF), lambda i: (i, 0)),
        pl.BlockSpec((HID, 1), lambda i: (0, 0)),
        pl.BlockSpec((1, 1), lambda i: (0, 0)),
    ],
    out_specs=[
        pl.BlockSpec((_B, HID), lambda i: (i, 0)),
        pl.BlockSpec((_B, 1), lambda i: (i, 0)),
    ],
    out_shape=[
        jax.ShapeDtypeStruct((NP, HID), jnp.float32),
        jax.ShapeDtypeStruct((NP, 1), jnp.float32),
    ],
)


def kernel(x_assignments, x_students, edge_index_sa, edge_index_as, params):
    p = params

    def pad_idx(e, fill):
        return jnp.concatenate(
            [e, jnp.full((EP - E,), fill, jnp.int32)]).reshape(CHT, EC)

    src_sa = pad_idx(edge_index_sa[0], 0)
    dst_sa = pad_idx(edge_index_sa[1], NP - 1)
    src_as = pad_idx(edge_index_as[0], 0)
    dst_as = pad_idx(edge_index_as[1], NP - 1)

    zrows = jnp.zeros((RPS, HALF), jnp.float32)
    zcnt = jnp.zeros((RPS, HALF), jnp.float32)
    ones16 = jnp.ones((EC, HALF), jnp.float32)

    xa = jnp.pad(x_assignments, ((0, NP - N), (0, 0)))
    xs = jnp.pad(x_students, ((0, NP - N), (0, 0)))
    ha_l, ha_r = _inproj(xa, p['in_W_a'], p['in_b_a'].reshape(1, HID))
    hs_l, hs_r = _inproj(xs, p['in_W_s'], p['in_b_s'].reshape(1, HID))

    cnt_a, cnt_s = _sc_edge_counts(dst_sa, dst_as, zcnt, ones16)

    for lp in p['layers']:
        agg_a_l, agg_a_r = _sc_edge_sum(hs_l, hs_r, src_sa, dst_sa, zrows)
        agg_s_l, agg_s_r = _sc_edge_sum(ha_l, ha_r, src_as, dst_as, zrows)
        ha_l, ha_r = _combine(agg_a_l, agg_a_r, cnt_a, ha_l, ha_r,
                              lp['sa_Wl'], lp['sa_bl'].reshape(1, HID),
                              lp['sa_Wr'], lp['ln_a_g'].reshape(1, HID),
                              lp['ln_a_b'].reshape(1, HID))
        hs_l, hs_r = _combine(agg_s_l, agg_s_r, cnt_s, hs_l, hs_r,
                              lp['as_Wl'], lp['as_bl'].reshape(1, HID),
                              lp['as_Wr'], lp['ln_s_g'].reshape(1, HID),
                              lp['ln_s_b'].reshape(1, HID))

    bias = (p['out_b'][0] + p['base']).reshape(1, 1)
    hidden, pred = _head(ha_l, ha_r, p['out_W'], bias)
    return (hidden[:N], pred[:N, 0])


# speedup vs baseline: 1.1658x; 1.1658x over previous
import functools

import jax
import jax.numpy as jnp
from jax import lax
from jax.experimental import pallas as pl
from jax.experimental.pallas import tpu as pltpu
from jax.experimental.pallas import tpu_sc as plsc

N = 10000
NP = 10240
E = 320000
D_IN = 128
HID = 256
HALF = 128

EC = 128
NSUB = 16
RPS = NP // NSUB
BLK = 16
NBLK = 10
CPS = BLK * NBLK
EP = CPS * NSUB * EC
CHT = EP // EC

_MESH = plsc.VectorSubcoreMesh(core_axis_name="c", subcore_axis_name="s")


@functools.partial(
    pl.kernel,
    out_type=(
        jax.ShapeDtypeStruct((NP, HALF), jnp.float32),
        jax.ShapeDtypeStruct((NP, HALF), jnp.float32),
    ),
    mesh=_MESH,
    scratch_types=(
        pltpu.VMEM((2, EC), jnp.int32),
        pltpu.VMEM((2, EC), jnp.int32),
        pltpu.VMEM((EC, HALF), jnp.float32),
        pltpu.VMEM((EC, HALF), jnp.float32),
        pltpu.VMEM_SHARED((NP, HALF), jnp.float32),
        pltpu.SemaphoreType.DMA,
    ),
)
def _sc_edge_sum(tab_l, tab_r, src2d, dst2d, zrows,
                 out_l, out_r, si2, di2, rows0, rows1, acc, gsem):
    c = lax.axis_index("c")
    s = lax.axis_index("s")
    slab = pl.ds(s * RPS, RPS)
    pltpu.sync_copy(zrows, acc.at[slab])
    plsc.subcore_barrier()

    def accumulate(tab):
        def step(i, carry):
            base = (i * NSUB + s) * 2
            pltpu.sync_copy(src2d.at[pl.ds(base, 2)], si2)
            pltpu.sync_copy(dst2d.at[pl.ds(base, 2)], di2)
            cg0 = pltpu.async_copy(tab.at[si2.at[0]], rows0, gsem)
            cg1 = pltpu.async_copy(tab.at[si2.at[1]], rows1, gsem)
            cg0.wait()
            cg1.wait()
            pltpu.sync_copy(rows0, acc.at[di2.at[0]], add=True)
            pltpu.sync_copy(rows1, acc.at[di2.at[1]], add=True)
            return carry

        lax.fori_loop(0, CPS // 2, step, 0)

    @pl.when(c == 0)
    def _():
        accumulate(tab_l)

    @pl.when(c == 1)
    def _():
        accumulate(tab_r)

    plsc.subcore_barrier()

    @pl.when(c == 0)
    def _():
        pltpu.sync_copy(acc.at[slab], out_l.at[slab])

    @pl.when(c == 1)
    def _():
        pltpu.sync_copy(acc.at[slab], out_r.at[slab])


@functools.partial(
    pl.kernel,
    out_type=(
        jax.ShapeDtypeStruct((NP, HALF), jnp.float32),
        jax.ShapeDtypeStruct((NP, HALF), jnp.float32),
    ),
    mesh=_MESH,
    scratch_types=(
        pltpu.VMEM((BLK, EC), jnp.int32),
        pltpu.VMEM((EC, HALF), jnp.float32),
        pltpu.VMEM_SHARED((NP, HALF), jnp.float32),
        pltpu.SemaphoreType.DMA,
    ),
)
def _sc_edge_counts(dst_sa, dst_as, zcnt, ones_hbm,
                    cnt_a, cnt_s, di_blk, ones_v, csh, sem):
    c = lax.axis_index("c")
    s = lax.axis_index("s")
    slab = pl.ds(s * RPS, RPS)
    pltpu.sync_copy(ones_hbm, ones_v)
    pltpu.sync_copy(zcnt, csh.at[slab])
    plsc.subcore_barrier()

    def accumulate(dst2d):
        def block(bi, carry):
            base = (s * NBLK + bi) * BLK
            pltpu.sync_copy(dst2d.at[pl.ds(base, BLK)], di_blk)
            cs = [pltpu.async_copy(ones_v, csh.at[di_blk.at[k]], sem,
                                   add=True)
                  for k in range(BLK)]
            for cp in cs:
                cp.wait()
            return carry

        lax.fori_loop(0, NBLK, block, 0)

    @pl.when(c == 0)
    def _():
        accumulate(dst_sa)

    @pl.when(c == 1)
    def _():
        accumulate(dst_as)

    plsc.subcore_barrier()

    @pl.when(c == 0)
    def _():
        pltpu.sync_copy(csh.at[slab], cnt_a.at[slab])

    @pl.when(c == 1)
    def _():
        pltpu.sync_copy(csh.at[slab], cnt_s.at[slab])


_B = 2048


def _dot(a, b):
    return jnp.dot(a.astype(jnp.bfloat16), b.astype(jnp.bfloat16),
                   preferred_element_type=jnp.float32)


def _inproj_body(x_ref, w_ref, b_ref, ol_ref, or_ref):
    h = jnp.maximum(_dot(x_ref[...], w_ref[...]) + b_ref[...], 0.0)
    ol_ref[...] = h[:, :HALF]
    or_ref[...] = h[:, HALF:]


_inproj = pl.pallas_call(
    _inproj_body,
    grid=(NP // _B,),
    in_specs=[
        pl.BlockSpec((_B, D_IN), lambda i: (i, 0)),
        pl.BlockSpec((D_IN, HID), lambda i: (0, 0)),
        pl.BlockSpec((1, HID), lambda i: (0, 0)),
    ],
    out_specs=[
        pl.BlockSpec((_B, HALF), lambda i: (i, 0)),
        pl.BlockSpec((_B, HALF), lambda i: (i, 0)),
    ],
    out_shape=[
        jax.ShapeDtypeStruct((NP, HALF), jnp.float32),
        jax.ShapeDtypeStruct((NP, HALF), jnp.float32),
    ],
)


def _combine_body(al_ref, ar_ref, cnt_ref, hl_ref, hr_ref,
                  wl_ref, bl_ref, wr_ref, g_ref, b_ref, ol_ref, or_ref):
    agg = jnp.concatenate([al_ref[...], ar_ref[...]], axis=1)
    h = jnp.concatenate([hl_ref[...], hr_ref[...]], axis=1)
    mean = agg / jnp.maximum(cnt_ref[...][:, :1], 1.0)
    new = _dot(mean, wl_ref[...]) + _dot(h, wr_ref[...]) + bl_ref[...]
    m = jnp.mean(new, axis=1, keepdims=True)
    v = jnp.mean((new - m) * (new - m), axis=1, keepdims=True)
    y = (new - m) / jnp.sqrt(v + 1e-5) * g_ref[...] + b_ref[...]
    y = jnp.maximum(y, 0.0)
    ol_ref[...] = y[:, :HALF]
    or_ref[...] = y[:, HALF:]


_combine = pl.pallas_call(
    _combine_body,
    grid=(NP // _B,),
    in_specs=[
        pl.BlockSpec((_B, HALF), lambda i: (i, 0)),
        pl.BlockSpec((_B, HALF), lambda i: (i, 0)),
        pl.BlockSpec((_B, HALF), lambda i: (i, 0)),
        pl.BlockSpec((_B, HALF), lambda i: (i, 0)),
        pl.BlockSpec((_B, HALF), lambda i: (i, 0)),
        pl.BlockSpec((HID, HID), lambda i: (0, 0)),
        pl.BlockSpec((1, HID), lambda i: (0, 0)),
        pl.BlockSpec((HID, HID), lambda i: (0, 0)),
        pl.BlockSpec((1, HID), lambda i: (0, 0)),
        pl.BlockSpec((1, HID), lambda i: (0, 0)),
    ],
    out_specs=[
        pl.BlockSpec((_B, HALF), lambda i: (i, 0)),
        pl.BlockSpec((_B, HALF), lambda i: (i, 0)),
    ],
    out_shape=[
        jax.ShapeDtypeStruct((NP, HALF), jnp.float32),
        jax.ShapeDtypeStruct((NP, HALF), jnp.float32),
    ],
)


def _head_body(hl_ref, hr_ref, w_ref, bias_ref, hid_ref, pred_ref):
    h = jnp.concatenate([hl_ref[...], hr_ref[...]], axis=1)
    hid_ref[...] = h
    wrow = jnp.reshape(w_ref[...], (1, HID))
    h16 = h.astype(jnp.bfloat16).astype(jnp.float32)
    w16 = wrow.astype(jnp.bfloat16).astype(jnp.float32)
    pred_ref[...] = (jnp.sum(h16 * w16, axis=1, keepdims=True)
                     + bias_ref[...])


_head = pl.pallas_call(
    _head_body,
    grid=(NP // _B,),
    in_specs=[
        pl.BlockSpec((_B, HALF), lambda i: (i, 0)),
        pl.BlockSpec((_B, HALF), lambda i: (i, 0)),
        pl.BlockSpec((HID, 1), lambda i: (0, 0)),
        pl.BlockSpec((1, 1), lambda i: (0, 0)),
    ],
    out_specs=[
        pl.BlockSpec((_B, HID), lambda i: (i, 0)),
        pl.BlockSpec((_B, 1), lambda i: (i, 0)),
    ],
    out_shape=[
        jax.ShapeDtypeStruct((NP, HID), jnp.float32),
        jax.ShapeDtypeStruct((NP, 1), jnp.float32),
    ],
)


def kernel(x_assignments, x_students, edge_index_sa, edge_index_as, params):
    p = params

    def pad_idx(e, fill):
        return jnp.concatenate(
            [e, jnp.full((EP - E,), fill, jnp.int32)]).reshape(CHT, EC)

    src_sa = pad_idx(edge_index_sa[0], 0)
    dst_sa = pad_idx(edge_index_sa[1], NP - 1)
    src_as = pad_idx(edge_index_as[0], 0)
    dst_as = pad_idx(edge_index_as[1], NP - 1)

    zrows = jnp.zeros((RPS, HALF), jnp.float32)
    zcnt = jnp.zeros((RPS, HALF), jnp.float32)
    ones16 = jnp.ones((EC, HALF), jnp.float32)

    xa = jnp.pad(x_assignments, ((0, NP - N), (0, 0)))
    xs = jnp.pad(x_students, ((0, NP - N), (0, 0)))
    ha_l, ha_r = _inproj(xa, p['in_W_a'], p['in_b_a'].reshape(1, HID))
    hs_l, hs_r = _inproj(xs, p['in_W_s'], p['in_b_s'].reshape(1, HID))

    cnt_a, cnt_s = _sc_edge_counts(dst_sa, dst_as, zcnt, ones16)

    for lp in p['layers']:
        agg_a_l, agg_a_r = _sc_edge_sum(hs_l, hs_r, src_sa, dst_sa, zrows)
        agg_s_l, agg_s_r = _sc_edge_sum(ha_l, ha_r, src_as, dst_as, zrows)
        ha_l, ha_r = _combine(agg_a_l, agg_a_r, cnt_a, ha_l, ha_r,
                              lp['sa_Wl'], lp['sa_bl'].reshape(1, HID),
                              lp['sa_Wr'], lp['ln_a_g'].reshape(1, HID),
                              lp['ln_a_b'].reshape(1, HID))
        hs_l, hs_r = _combine(agg_s_l, agg_s_r, cnt_s, hs_l, hs_r,
                              lp['as_Wl'], lp['as_bl'].reshape(1, HID),
                              lp['as_Wr'], lp['ln_s_g'].reshape(1, HID),
                              lp['ln_s_b'].reshape(1, HID))

    bias = (p['out_b'][0] + p['base']).reshape(1, 1)
    hidden, pred = _head(ha_l, ha_r, p['out_W'], bias)
    return (hidden[:N], pred[:N, 0])


# speedup vs baseline: 1.8871x; 1.6188x over previous
import functools

import jax
import jax.numpy as jnp
from jax import lax
from jax.experimental import pallas as pl
from jax.experimental.pallas import tpu as pltpu
from jax.experimental.pallas import tpu_sc as plsc

N = 10000
NP = 10240
E = 320000
D_IN = 128
HID = 256
HALF = 128

EC = 128
SUP = 2
NSUP = (E // EC) // SUP
NSUB = 16
RPS = NP // NSUB
BLK = 16
NBLK = 10
CPS = BLK * NBLK
EP = CPS * NSUB * EC
CHT = EP // EC

_MESH = plsc.VectorSubcoreMesh(core_axis_name="c", subcore_axis_name="s")


@functools.partial(
    pl.kernel,
    out_type=(
        jax.ShapeDtypeStruct((NP, HALF), jnp.float32),
        jax.ShapeDtypeStruct((NP, HALF), jnp.float32),
    ),
    mesh=_MESH,
    scratch_types=(
        pltpu.VMEM((SUP, EC), jnp.int32),
        pltpu.VMEM((SUP, EC), jnp.int32),
        pltpu.VMEM((SUP * EC, HALF), jnp.float32),
        pltpu.VMEM_SHARED((NP, HALF), jnp.float32),
        pltpu.SemaphoreType.DMA,
    ),
)
def _sc_edge_sum(tab_l, tab_r, src2d, dst2d, zrows,
                 out_l, out_r, si_v, di_v, rows_v, acc, gsem):
    c = lax.axis_index("c")
    s = lax.axis_index("s")
    slab = pl.ds(s * RPS, RPS)
    pltpu.sync_copy(zrows, acc.at[slab])
    plsc.subcore_barrier()

    iters = (NSUP - s + NSUB - 1) // NSUB

    def accumulate(tab):
        def step(i, carry):
            base = (i * NSUB + s) * SUP
            pltpu.sync_copy(src2d.at[pl.ds(base, SUP)], si_v)
            pltpu.sync_copy(dst2d.at[pl.ds(base, SUP)], di_v)
            cps = [
                pltpu.async_copy(tab.at[si_v.at[j]],
                                 rows_v.at[pl.ds(j * EC, EC)], gsem)
                for j in range(SUP)
            ]
            for cp in cps:
                cp.wait()
            for j in range(SUP):
                pltpu.sync_copy(rows_v.at[pl.ds(j * EC, EC)],
                                acc.at[di_v.at[j]], add=True)
            return carry

        lax.fori_loop(0, iters, step, 0)

    @pl.when(c == 0)
    def _():
        accumulate(tab_l)

    @pl.when(c == 1)
    def _():
        accumulate(tab_r)

    plsc.subcore_barrier()

    @pl.when(c == 0)
    def _():
        pltpu.sync_copy(acc.at[slab], out_l.at[slab])

    @pl.when(c == 1)
    def _():
        pltpu.sync_copy(acc.at[slab], out_r.at[slab])


@functools.partial(
    pl.kernel,
    out_type=(
        jax.ShapeDtypeStruct((NP, HALF), jnp.float32),
        jax.ShapeDtypeStruct((NP, HALF), jnp.float32),
    ),
    mesh=_MESH,
    scratch_types=(
        pltpu.VMEM((BLK, EC), jnp.int32),
        pltpu.VMEM((EC, HALF), jnp.float32),
        pltpu.VMEM_SHARED((NP, HALF), jnp.float32),
        pltpu.SemaphoreType.DMA,
    ),
)
def _sc_edge_counts(dst_sa, dst_as, zcnt, ones_hbm,
                    cnt_a, cnt_s, di_blk, ones_v, csh, sem):
    c = lax.axis_index("c")
    s = lax.axis_index("s")
    slab = pl.ds(s * RPS, RPS)
    pltpu.sync_copy(ones_hbm, ones_v)
    pltpu.sync_copy(zcnt, csh.at[slab])
    plsc.subcore_barrier()

    def accumulate(dst2d):
        def block(bi, carry):
            base = (s * NBLK + bi) * BLK
            pltpu.sync_copy(dst2d.at[pl.ds(base, BLK)], di_blk)
            cs = [pltpu.async_copy(ones_v, csh.at[di_blk.at[k]], sem,
                                   add=True)
                  for k in range(BLK)]
            for cp in cs:
                cp.wait()
            return carry

        lax.fori_loop(0, NBLK, block, 0)

    @pl.when(c == 0)
    def _():
        accumulate(dst_sa)

    @pl.when(c == 1)
    def _():
        accumulate(dst_as)

    plsc.subcore_barrier()

    @pl.when(c == 0)
    def _():
        pltpu.sync_copy(csh.at[slab], cnt_a.at[slab])

    @pl.when(c == 1)
    def _():
        pltpu.sync_copy(csh.at[slab], cnt_s.at[slab])


_B = 2048


def _dot(a, b):
    return jnp.dot(a.astype(jnp.bfloat16), b.astype(jnp.bfloat16),
                   preferred_element_type=jnp.float32)


def _inproj_body(x_ref, w_ref, b_ref, ol_ref, or_ref):
    h = jnp.maximum(_dot(x_ref[...], w_ref[...]) + b_ref[...], 0.0)
    ol_ref[...] = h[:, :HALF]
    or_ref[...] = h[:, HALF:]


_inproj = pl.pallas_call(
    _inproj_body,
    grid=(NP // _B,),
    in_specs=[
        pl.BlockSpec((_B, D_IN), lambda i: (i, 0)),
        pl.BlockSpec((D_IN, HID), lambda i: (0, 0)),
        pl.BlockSpec((1, HID), lambda i: (0, 0)),
    ],
    out_specs=[
        pl.BlockSpec((_B, HALF), lambda i: (i, 0)),
        pl.BlockSpec((_B, HALF), lambda i: (i, 0)),
    ],
    out_shape=[
        jax.ShapeDtypeStruct((NP, HALF), jnp.float32),
        jax.ShapeDtypeStruct((NP, HALF), jnp.float32),
    ],
)


def _combine_body(al_ref, ar_ref, cnt_ref, hl_ref, hr_ref,
                  wl_ref, bl_ref, wr_ref, g_ref, b_ref, ol_ref, or_ref):
    agg = jnp.concatenate([al_ref[...], ar_ref[...]], axis=1)
    h = jnp.concatenate([hl_ref[...], hr_ref[...]], axis=1)
    mean = agg / jnp.maximum(cnt_ref[...][:, :1], 1.0)
    new = _dot(mean, wl_ref[...]) + _dot(h, wr_ref[...]) + bl_ref[...]
    m = jnp.mean(new, axis=1, keepdims=True)
    v = jnp.mean((new - m) * (new - m), axis=1, keepdims=True)
    y = (new - m) / jnp.sqrt(v + 1e-5) * g_ref[...] + b_ref[...]
    y = jnp.maximum(y, 0.0)
    ol_ref[...] = y[:, :HALF]
    or_ref[...] = y[:, HALF:]


_combine = pl.pallas_call(
    _combine_body,
    grid=(NP // _B,),
    in_specs=[
        pl.BlockSpec((_B, HALF), lambda i: (i, 0)),
        pl.BlockSpec((_B, HALF), lambda i: (i, 0)),
        pl.BlockSpec((_B, HALF), lambda i: (i, 0)),
        pl.BlockSpec((_B, HALF), lambda i: (i, 0)),
        pl.BlockSpec((_B, HALF), lambda i: (i, 0)),
        pl.BlockSpec((HID, HID), lambda i: (0, 0)),
        pl.BlockSpec((1, HID), lambda i: (0, 0)),
        pl.BlockSpec((HID, HID), lambda i: (0, 0)),
        pl.BlockSpec((1, HID), lambda i: (0, 0)),
        pl.BlockSpec((1, HID), lambda i: (0, 0)),
    ],
    out_specs=[
        pl.BlockSpec((_B, HALF), lambda i: (i, 0)),
        pl.BlockSpec((_B, HALF), lambda i: (i, 0)),
    ],
    out_shape=[
        jax.ShapeDtypeStruct((NP, HALF), jnp.float32),
        jax.ShapeDtypeStruct((NP, HALF), jnp.float32),
    ],
)


def _head_body(hl_ref, hr_ref, w_ref, bias_ref, hid_ref, pred_ref):
    h = jnp.concatenate([hl_ref[...], hr_ref[...]], axis=1)
    hid_ref[...] = h
    wrow = jnp.reshape(w_ref[...], (1, HID))
    h16 = h.astype(jnp.bfloat16).astype(jnp.float32)
    w16 = wrow.astype(jnp.bfloat16).astype(jnp.float32)
    pred_ref[...] = (jnp.sum(h16 * w16, axis=1, keepdims=True)
                     + bias_ref[...])


_head = pl.pallas_call(
    _head_body,
    grid=(NP // _B,),
    in_specs=[
        pl.BlockSpec((_B, HALF), lambda i: (i, 0)),
        pl.BlockSpec((_B, HALF), lambda i: (i, 0)),
        pl.BlockSpec((HID, 1), lambda i: (0, 0)),
        pl.BlockSpec((1, 1), lambda i: (0, 0)),
    ],
    out_specs=[
        pl.BlockSpec((_B, HID), lambda i: (i, 0)),
        pl.BlockSpec((_B, 1), lambda i: (i, 0)),
    ],
    out_shape=[
        jax.ShapeDtypeStruct((NP, HID), jnp.float32),
        jax.ShapeDtypeStruct((NP, 1), jnp.float32),
    ],
)


def kernel(x_assignments, x_students, edge_index_sa, edge_index_as, params):
    p = params

    def pad_idx(e, fill):
        return jnp.concatenate(
            [e, jnp.full((EP - E,), fill, jnp.int32)]).reshape(CHT, EC)

    src_sa = edge_index_sa[0].reshape(E // EC, EC)
    dst_sa = edge_index_sa[1].reshape(E // EC, EC)
    src_as = edge_index_as[0].reshape(E // EC, EC)
    dst_as = edge_index_as[1].reshape(E // EC, EC)
    dst_sa_p = pad_idx(edge_index_sa[1], NP - 1)
    dst_as_p = pad_idx(edge_index_as[1], NP - 1)

    zrows = jnp.zeros((RPS, HALF), jnp.float32)
    zcnt = jnp.zeros((RPS, HALF), jnp.float32)
    ones16 = jnp.ones((EC, HALF), jnp.float32)

    xa = jnp.pad(x_assignments, ((0, NP - N), (0, 0)))
    xs = jnp.pad(x_students, ((0, NP - N), (0, 0)))
    ha_l, ha_r = _inproj(xa, p['in_W_a'], p['in_b_a'].reshape(1, HID))
    hs_l, hs_r = _inproj(xs, p['in_W_s'], p['in_b_s'].reshape(1, HID))

    cnt_a, cnt_s = _sc_edge_counts(dst_sa_p, dst_as_p, zcnt, ones16)

    for lp in p['layers']:
        agg_a_l, agg_a_r = _sc_edge_sum(hs_l, hs_r, src_sa, dst_sa, zrows)
        agg_s_l, agg_s_r = _sc_edge_sum(ha_l, ha_r, src_as, dst_as, zrows)
        ha_l, ha_r = _combine(agg_a_l, agg_a_r, cnt_a, ha_l, ha_r,
                              lp['sa_Wl'], lp['sa_bl'].reshape(1, HID),
                              lp['sa_Wr'], lp['ln_a_g'].reshape(1, HID),
                              lp['ln_a_b'].reshape(1, HID))
        hs_l, hs_r = _combine(agg_s_l, agg_s_r, cnt_s, hs_l, hs_r,
                              lp['as_Wl'], lp['as_bl'].reshape(1, HID),
                              lp['as_Wr'], lp['ln_s_g'].reshape(1, HID),
                              lp['ln_s_b'].reshape(1, HID))

    bias = (p['out_b'][0] + p['base']).reshape(1, 1)
    hidden, pred = _head(ha_l, ha_r, p['out_W'], bias)
    return (hidden[:N], pred[:N, 0])


# speedup vs baseline: 2.0561x; 1.0895x over previous
import functools

import jax
import jax.numpy as jnp
from jax import lax
from jax.experimental import pallas as pl
from jax.experimental.pallas import tpu as pltpu
from jax.experimental.pallas import tpu_sc as plsc

N = 10000
NP = 10240
E = 320000
D_IN = 128
HID = 256
HALF = 128

EC = 128
SUP = 2
NSUP = (E // EC) // SUP
NSUB = 16
RPS = NP // NSUB
BLK = 16
NBLK = 10
CPS = BLK * NBLK
EP = CPS * NSUB * EC
CHT = EP // EC

_MESH = plsc.VectorSubcoreMesh(core_axis_name="c", subcore_axis_name="s")


@functools.partial(
    pl.kernel,
    out_type=(
        jax.ShapeDtypeStruct((NP, HALF), jnp.float32),
        jax.ShapeDtypeStruct((NP, HALF), jnp.float32),
    ),
    mesh=_MESH,
    scratch_types=(
        pltpu.VMEM((SUP, 2, EC), jnp.int32),
        pltpu.VMEM((SUP * EC, HALF), jnp.float32),
        pltpu.VMEM_SHARED((NP, HALF), jnp.float32),
        pltpu.SemaphoreType.DMA,
    ),
)
def _sc_edge_sum(tab_l, tab_r, sd3d, zrows,
                 out_l, out_r, idx_v, rows_v, acc, gsem):
    c = lax.axis_index("c")
    s = lax.axis_index("s")
    slab = pl.ds(s * RPS, RPS)
    pltpu.sync_copy(zrows, acc.at[slab])
    plsc.subcore_barrier()

    iters = (NSUP - s + NSUB - 1) // NSUB

    def accumulate(tab):
        def step(i, carry):
            base = (i * NSUB + s) * SUP
            pltpu.sync_copy(sd3d.at[pl.ds(base, SUP)], idx_v)
            cps = [
                pltpu.async_copy(tab.at[idx_v.at[j, 0]],
                                 rows_v.at[pl.ds(j * EC, EC)], gsem)
                for j in range(SUP)
            ]
            for cp in cps:
                cp.wait()
            for j in range(SUP):
                pltpu.sync_copy(rows_v.at[pl.ds(j * EC, EC)],
                                acc.at[idx_v.at[j, 1]], add=True)
            return carry

        lax.fori_loop(0, iters, step, 0)

    @pl.when(c == 0)
    def _():
        accumulate(tab_l)

    @pl.when(c == 1)
    def _():
        accumulate(tab_r)

    plsc.subcore_barrier()

    @pl.when(c == 0)
    def _():
        pltpu.sync_copy(acc.at[slab], out_l.at[slab])

    @pl.when(c == 1)
    def _():
        pltpu.sync_copy(acc.at[slab], out_r.at[slab])


@functools.partial(
    pl.kernel,
    out_type=(
        jax.ShapeDtypeStruct((NP, HALF), jnp.float32),
        jax.ShapeDtypeStruct((NP, HALF), jnp.float32),
    ),
    mesh=_MESH,
    scratch_types=(
        pltpu.VMEM((BLK, EC), jnp.int32),
        pltpu.VMEM((EC, HALF), jnp.float32),
        pltpu.VMEM_SHARED((NP, HALF), jnp.float32),
        pltpu.SemaphoreType.DMA,
    ),
)
def _sc_edge_counts(dst_sa, dst_as, zcnt, ones_hbm,
                    cnt_a, cnt_s, di_blk, ones_v, csh, sem):
    c = lax.axis_index("c")
    s = lax.axis_index("s")
    slab = pl.ds(s * RPS, RPS)
    pltpu.sync_copy(ones_hbm, ones_v)
    pltpu.sync_copy(zcnt, csh.at[slab])
    plsc.subcore_barrier()

    def accumulate(dst2d):
        def block(bi, carry):
            base = (s * NBLK + bi) * BLK
            pltpu.sync_copy(dst2d.at[pl.ds(base, BLK)], di_blk)
            cs = [pltpu.async_copy(ones_v, csh.at[di_blk.at[k]], sem,
                                   add=True)
                  for k in range(BLK)]
            for cp in cs:
                cp.wait()
            return carry

        lax.fori_loop(0, NBLK, block, 0)

    @pl.when(c == 0)
    def _():
        accumulate(dst_sa)

    @pl.when(c == 1)
    def _():
        accumulate(dst_as)

    plsc.subcore_barrier()

    @pl.when(c == 0)
    def _():
        pltpu.sync_copy(csh.at[slab], cnt_a.at[slab])

    @pl.when(c == 1)
    def _():
        pltpu.sync_copy(csh.at[slab], cnt_s.at[slab])


_B = 2048


def _dot(a, b):
    return jnp.dot(a.astype(jnp.bfloat16), b.astype(jnp.bfloat16),
                   preferred_element_type=jnp.float32)


def _inproj_body(x_ref, w_ref, b_ref, ol_ref, or_ref):
    h = jnp.maximum(_dot(x_ref[...], w_ref[...]) + b_ref[...], 0.0)
    ol_ref[...] = h[:, :HALF]
    or_ref[...] = h[:, HALF:]


_inproj = pl.pallas_call(
    _inproj_body,
    grid=(NP // _B,),
    in_specs=[
        pl.BlockSpec((_B, D_IN), lambda i: (i, 0)),
        pl.BlockSpec((D_IN, HID), lambda i: (0, 0)),
        pl.BlockSpec((1, HID), lambda i: (0, 0)),
    ],
    out_specs=[
        pl.BlockSpec((_B, HALF), lambda i: (i, 0)),
        pl.BlockSpec((_B, HALF), lambda i: (i, 0)),
    ],
    out_shape=[
        jax.ShapeDtypeStruct((NP, HALF), jnp.float32),
        jax.ShapeDtypeStruct((NP, HALF), jnp.float32),
    ],
)


def _combine_body(al_ref, ar_ref, cnt_ref, hl_ref, hr_ref,
                  wl_ref, bl_ref, wr_ref, g_ref, b_ref, ol_ref, or_ref):
    agg = jnp.concatenate([al_ref[...], ar_ref[...]], axis=1)
    h = jnp.concatenate([hl_ref[...], hr_ref[...]], axis=1)
    mean = agg / jnp.maximum(cnt_ref[...][:, :1], 1.0)
    new = _dot(mean, wl_ref[...]) + _dot(h, wr_ref[...]) + bl_ref[...]
    m = jnp.mean(new, axis=1, keepdims=True)
    v = jnp.mean((new - m) * (new - m), axis=1, keepdims=True)
    y = (new - m) / jnp.sqrt(v + 1e-5) * g_ref[...] + b_ref[...]
    y = jnp.maximum(y, 0.0)
    ol_ref[...] = y[:, :HALF]
    or_ref[...] = y[:, HALF:]


_combine = pl.pallas_call(
    _combine_body,
    grid=(NP // _B,),
    in_specs=[
        pl.BlockSpec((_B, HALF), lambda i: (i, 0)),
        pl.BlockSpec((_B, HALF), lambda i: (i, 0)),
        pl.BlockSpec((_B, HALF), lambda i: (i, 0)),
        pl.BlockSpec((_B, HALF), lambda i: (i, 0)),
        pl.BlockSpec((_B, HALF), lambda i: (i, 0)),
        pl.BlockSpec((HID, HID), lambda i: (0, 0)),
        pl.BlockSpec((1, HID), lambda i: (0, 0)),
        pl.BlockSpec((HID, HID), lambda i: (0, 0)),
        pl.BlockSpec((1, HID), lambda i: (0, 0)),
        pl.BlockSpec((1, HID), lambda i: (0, 0)),
    ],
    out_specs=[
        pl.BlockSpec((_B, HALF), lambda i: (i, 0)),
        pl.BlockSpec((_B, HALF), lambda i: (i, 0)),
    ],
    out_shape=[
        jax.ShapeDtypeStruct((NP, HALF), jnp.float32),
        jax.ShapeDtypeStruct((NP, HALF), jnp.float32),
    ],
)


def _head_body(hl_ref, hr_ref, w_ref, bias_ref, hid_ref, pred_ref):
    h = jnp.concatenate([hl_ref[...], hr_ref[...]], axis=1)
    hid_ref[...] = h
    wrow = jnp.reshape(w_ref[...], (1, HID))
    h16 = h.astype(jnp.bfloat16).astype(jnp.float32)
    w16 = wrow.astype(jnp.bfloat16).astype(jnp.float32)
    pred_ref[...] = (jnp.sum(h16 * w16, axis=1, keepdims=True)
                     + bias_ref[...])


_head = pl.pallas_call(
    _head_body,
    grid=(NP // _B,),
    in_specs=[
        pl.BlockSpec((_B, HALF), lambda i: (i, 0)),
        pl.BlockSpec((_B, HALF), lambda i: (i, 0)),
        pl.BlockSpec((HID, 1), lambda i: (0, 0)),
        pl.BlockSpec((1, 1), lambda i: (0, 0)),
    ],
    out_specs=[
        pl.BlockSpec((_B, HID), lambda i: (i, 0)),
        pl.BlockSpec((_B, 1), lambda i: (i, 0)),
    ],
    out_shape=[
        jax.ShapeDtypeStruct((NP, HID), jnp.float32),
        jax.ShapeDtypeStruct((NP, 1), jnp.float32),
    ],
)


def kernel(x_assignments, x_students, edge_index_sa, edge_index_as, params):
    p = params

    def pad_idx(e, fill):
        return jnp.concatenate(
            [e, jnp.full((EP - E,), fill, jnp.int32)]).reshape(CHT, EC)

    sd_sa = jnp.stack([edge_index_sa[0].reshape(E // EC, EC),
                       edge_index_sa[1].reshape(E // EC, EC)], axis=1)
    sd_as = jnp.stack([edge_index_as[0].reshape(E // EC, EC),
                       edge_index_as[1].reshape(E // EC, EC)], axis=1)
    dst_sa_p = pad_idx(edge_index_sa[1], NP - 1)
    dst_as_p = pad_idx(edge_index_as[1], NP - 1)

    zrows = jnp.zeros((RPS, HALF), jnp.float32)
    zcnt = jnp.zeros((RPS, HALF), jnp.float32)
    ones16 = jnp.ones((EC, HALF), jnp.float32)

    xa = jnp.pad(x_assignments, ((0, NP - N), (0, 0)))
    xs = jnp.pad(x_students, ((0, NP - N), (0, 0)))
    ha_l, ha_r = _inproj(xa, p['in_W_a'], p['in_b_a'].reshape(1, HID))
    hs_l, hs_r = _inproj(xs, p['in_W_s'], p['in_b_s'].reshape(1, HID))

    cnt_a, cnt_s = _sc_edge_counts(dst_sa_p, dst_as_p, zcnt, ones16)

    for lp in p['layers']:
        agg_a_l, agg_a_r = _sc_edge_sum(hs_l, hs_r, sd_sa, zrows)
        agg_s_l, agg_s_r = _sc_edge_sum(ha_l, ha_r, sd_as, zrows)
        ha_l, ha_r = _combine(agg_a_l, agg_a_r, cnt_a, ha_l, ha_r,
                              lp['sa_Wl'], lp['sa_bl'].reshape(1, HID),
                              lp['sa_Wr'], lp['ln_a_g'].reshape(1, HID),
                              lp['ln_a_b'].reshape(1, HID))
        hs_l, hs_r = _combine(agg_s_l, agg_s_r, cnt_s, hs_l, hs_r,
                              lp['as_Wl'], lp['as_bl'].reshape(1, HID),
                              lp['as_Wr'], lp['ln_s_g'].reshape(1, HID),
                              lp['ln_s_b'].reshape(1, HID))

    bias = (p['out_b'][0] + p['base']).reshape(1, 1)
    hidden, pred = _head(ha_l, ha_r, p['out_W'], bias)
    return (hidden[:N], pred[:N, 0])
